# baseline (speedup 1.0000x reference)
import functools

import jax
import jax.numpy as jnp
from jax import lax
from jax.experimental import pallas as pl
from jax.experimental.pallas import tpu as pltpu
from jax.experimental.pallas import tpu_sc as plsc

_B = 1024
_K = 50
_V = 100000
_N = _B * _K

_NC = 2
_NS = 16
_NW = _NC * _NS
_PW = _N // _NW
_G = _PW // 16
_DMA_LEN = 80
_NDMA = _PW // _DMA_LEN


def _body(flat_in, idx_in, idx_out, val_out, idx_v, fidx_v, out_v, sem):
    c = lax.axis_index("c")
    s = lax.axis_index("s")
    wid = s * _NC + c
    base = wid * _PW

    pltpu.sync_copy(idx_in.at[pl.ds(base, _PW)], idx_v)

    lane = lax.iota(jnp.int32, 16)

    def compute(g, carry):
        off = g * 16
        idx16 = idx_v[pl.ds(off, 16)]
        start = (base + off).astype(jnp.int32)
        pos = lax.add(lane, jax.lax.broadcast(start, (16,)))
        row = lax.div(pos, jnp.full((16,), _K, jnp.int32))
        fidx_v[pl.ds(off, 16)] = lax.add(
            lax.mul(row, jnp.full((16,), _V, jnp.int32)), idx16)
        return carry

    lax.fori_loop(0, _G, compute, 0)

    copies = []
    for r in range(_NDMA):
        o = r * _DMA_LEN
        copies.append(pltpu.async_copy(
            flat_in.at[fidx_v.at[pl.ds(o, _DMA_LEN)]],
            out_v.at[pl.ds(o, _DMA_LEN)], sem))
    for cp in copies:
        cp.wait()

    pltpu.sync_copy(idx_v, idx_out.at[pl.ds(base, _PW)])
    pltpu.sync_copy(out_v, val_out.at[pl.ds(base, _PW)])


@jax.jit
def _run(flat_in, idx_flat):
    mesh = plsc.VectorSubcoreMesh(core_axis_name="c", subcore_axis_name="s")
    f = functools.partial(
        pl.kernel, mesh=mesh,
        out_type=[jax.ShapeDtypeStruct((_N,), jnp.int32),
                  jax.ShapeDtypeStruct((_N,), jnp.float32)],
        scratch_types=[
            pltpu.VMEM((_PW,), jnp.int32),
            pltpu.VMEM((_PW,), jnp.int32),
            pltpu.VMEM((_PW,), jnp.float32),
            pltpu.SemaphoreType.DMA,
        ])(_body)
    return f(flat_in, idx_flat)


def kernel(inputs, indices):
    idx32 = indices.astype(jnp.int32)
    idx_o, val_o = _run(inputs.reshape(-1), idx32.reshape(-1))
    return idx_o.reshape(indices.shape), val_o.reshape(indices.shape)


# speedup vs baseline: 2.2593x; 2.2593x over previous
import functools

import jax
import jax.numpy as jnp
from jax import lax
from jax.experimental import pallas as pl
from jax.experimental.pallas import tpu as pltpu
from jax.experimental.pallas import tpu_sc as plsc

_B = 1024
_K = 50
_N = _B * _K

_NC = 2
_NS = 16
_NW = _NC * _NS
_RW = _B // _NW
_PW = _N // _NW
_G = _PW // 16
_CH = 800


def _body(in_ref, idx_ref, idx_out, val_out, idx_v, pout_v, out_v, sem):
    c = lax.axis_index("c")
    s = lax.axis_index("s")
    wid = s * _NC + c
    base = wid * _PW
    row0 = wid * _RW

    pltpu.sync_copy(idx_ref.at[pl.ds(base, _PW)], idx_v)

    lane = lax.iota(jnp.int32, 16)

    for p in range(_PW // _CH):
        i_base = p * _CH

        def fire(g, carry):
            i0 = i_base + g * 16
            offs = lax.add(lane, jnp.full((16,), i0, jnp.int32))
            v16 = plsc.load_gather(idx_v, [offs])
            for j in range(16):
                i = i0 + j
                v = v16[j]
                b = row0 + lax.div(i, _K)
                c0 = lax.mul(lax.div(v, 128), 128)
                pltpu.async_copy(
                    in_ref.at[b, pl.ds(c0, 128)],
                    pout_v.at[g * 16 + j], sem)
            return carry

        lax.fori_loop(0, _CH // 16, fire, 0)

        pltpu.make_async_copy(
            in_ref.at[pl.ds(0, _CH), pl.ds(0, 128)], pout_v, sem).wait()

        for g in range(_CH // 16):
            rows = lax.add(lane, jnp.full((16,), g * 16, jnp.int32))
            v16 = plsc.load_gather(idx_v, [lax.add(rows, jnp.full((16,), i_base, jnp.int32))])
            lanes = lax.rem(v16, jnp.full((16,), 128, jnp.int32))
            out_v[pl.ds(i_base + g * 16, 16)] = plsc.load_gather(
                pout_v, [rows, lanes])

    pltpu.sync_copy(idx_v, idx_out.at[pl.ds(base, _PW)])
    pltpu.sync_copy(out_v, val_out.at[pl.ds(base, _PW)])


@jax.jit
def _run(inputs, idx_flat):
    mesh = plsc.VectorSubcoreMesh(core_axis_name="c", subcore_axis_name="s")
    f = functools.partial(
        pl.kernel, mesh=mesh,
        out_type=[jax.ShapeDtypeStruct((_N,), jnp.int32),
                  jax.ShapeDtypeStruct((_N,), jnp.float32)],
        scratch_types=[
            pltpu.VMEM((_PW,), jnp.int32),
            pltpu.VMEM((_CH, 128), jnp.float32),
            pltpu.VMEM((_PW,), jnp.float32),
            pltpu.SemaphoreType.DMA,
        ],
        compiler_params=pltpu.CompilerParams(needs_layout_passes=False),
    )(_body)
    return f(inputs, idx_flat)


def kernel(inputs, indices):
    idx32 = indices.astype(jnp.int32)
    idx_o, val_o = _run(inputs, idx32.reshape(-1))
    return idx_o.reshape(indices.shape), val_o.reshape(indices.shape)
